# baseline (device time: 228368 ns/iter reference)
import jax
import jax.numpy as jnp
from jax import lax
from jax.experimental import pallas as pl
from jax.experimental.pallas import tpu as pltpu

N_DEV = 16
B, S, D = 2, 512, 2048
DC = 128
H, DH, DR = 16, 128, 32
T = B * S
SCALE = (DH + DR) ** -0.5


def _body(x_ref, wdkv_ref, wuk_ref, wuv_ref, wq_ref, wqr_ref, wkr_ref,
          wo_ref, out_ref, stage_ref, rs_recv_ref, o_ref, ag_recv_ref,
          rs_send_sems, rs_recv_sems, ag_send_sems, ag_recv_sems):
    my = lax.axis_index("i")
    left = lax.rem(my + N_DEV - 1, N_DEV)
    right = lax.rem(my + 1, N_DEV)

    bsem = pltpu.get_barrier_semaphore()
    for nbr in (left, right):
        pl.semaphore_signal(bsem, inc=1, device_id=(nbr,),
                            device_id_type=pl.DeviceIdType.MESH)
    pl.semaphore_wait(bsem, 2)

    x = x_ref[...]
    c = jnp.dot(x, wdkv_ref[...],
                preferred_element_type=jnp.float32).astype(jnp.bfloat16)

    def local_chunk(j):
        kj = jnp.dot(c, wuk_ref[:, pl.ds(j * DH, DH)],
                     preferred_element_type=jnp.float32)
        vj = jnp.dot(c, wuv_ref[:, pl.ds(j * DH, DH)],
                     preferred_element_type=jnp.float32)
        return jnp.concatenate([kj, vj], axis=1)

    rs_rdmas = []
    for t in range(N_DEV - 1):
        p = t % 2
        if t >= 2:
            rs_rdmas[t - 2].wait_send()
        sj = lax.rem(my + N_DEV - t - 1, N_DEV)
        acc = local_chunk(sj)
        if t >= 1:
            rs_rdmas[t - 1].wait_recv()
            acc = acc + rs_recv_ref[t - 1].astype(jnp.float32)
        stage_ref[p, :, :] = acc.astype(jnp.bfloat16)
        rdma = pltpu.make_async_remote_copy(
            src_ref=stage_ref.at[p],
            dst_ref=rs_recv_ref.at[t],
            send_sem=rs_send_sems.at[p],
            recv_sem=rs_recv_sems.at[t],
            device_id=(right,),
            device_id_type=pl.DeviceIdType.MESH,
        )
        rdma.start()
        rs_rdmas.append(rdma)

    rs_rdmas[N_DEV - 2].wait_recv()
    kv = local_chunk(my) + rs_recv_ref[N_DEV - 2].astype(jnp.float32)
    k_h = kv[:, :DH].astype(jnp.bfloat16)
    v_h = kv[:, DH:].astype(jnp.bfloat16)

    q = jnp.dot(x, wq_ref[...],
                preferred_element_type=jnp.float32).astype(jnp.bfloat16)
    qr = jnp.dot(x, wqr_ref[...],
                 preferred_element_type=jnp.float32).astype(jnp.bfloat16)
    kr = jnp.dot(x, wkr_ref[...],
                 preferred_element_type=jnp.float32).astype(jnp.bfloat16)
    for b in range(B):
        sl = slice(b * S, (b + 1) * S)
        s1 = lax.dot_general(q[sl], k_h[sl], (((1,), (1,)), ((), ())),
                             preferred_element_type=jnp.float32)
        s2 = lax.dot_general(qr[sl], kr[sl], (((1,), (1,)), ((), ())),
                             preferred_element_type=jnp.float32)
        sc = (s1 + s2) * SCALE
        mx = jnp.max(sc, axis=1, keepdims=True)
        e = jnp.exp(sc - mx)
        pmat = (e / jnp.sum(e, axis=1, keepdims=True)).astype(jnp.bfloat16)
        ob = jnp.dot(pmat, v_h[sl], preferred_element_type=jnp.float32)
        o_ref[pl.ds(b * S, S), :] = ob.astype(jnp.bfloat16)

    def wo_rows(j):
        return wo_ref[pl.ds(j * DH, DH), :]

    ag_rdmas = []
    first = pltpu.make_async_remote_copy(
        src_ref=o_ref,
        dst_ref=ag_recv_ref.at[0],
        send_sem=ag_send_sems.at[0],
        recv_sem=ag_recv_sems.at[0],
        device_id=(right,),
        device_id_type=pl.DeviceIdType.MESH,
    )
    first.start()
    ag_rdmas.append(first)
    out_ref[...] = jnp.dot(o_ref[...], wo_rows(my),
                           preferred_element_type=jnp.float32)
    for t in range(1, N_DEV - 1):
        ag_rdmas[t - 1].wait_recv()
        rdma = pltpu.make_async_remote_copy(
            src_ref=ag_recv_ref.at[t - 1],
            dst_ref=ag_recv_ref.at[t],
            send_sem=ag_send_sems.at[t],
            recv_sem=ag_recv_sems.at[t],
            device_id=(right,),
            device_id_type=pl.DeviceIdType.MESH,
        )
        rdma.start()
        ag_rdmas.append(rdma)
        j = lax.rem(my + N_DEV - t, N_DEV)
        out_ref[...] += jnp.dot(ag_recv_ref[t - 1], wo_rows(j),
                                preferred_element_type=jnp.float32)
    ag_rdmas[N_DEV - 2].wait_recv()
    out_ref[...] += jnp.dot(ag_recv_ref[N_DEV - 2], wo_rows(right),
                            preferred_element_type=jnp.float32)

    rs_rdmas[N_DEV - 3].wait_send()
    rs_rdmas[N_DEV - 2].wait_send()
    for rdma in ag_rdmas:
        rdma.wait_send()


def kernel(x, Wdkv, Wuk, Wuv, Wq, Wqr, Wkr, Wo):
    my = lax.axis_index("i")
    xb = x.reshape(T, D).astype(jnp.bfloat16)
    wq_h = lax.dynamic_slice(Wq, (0, my * DH), (D, DH)).astype(jnp.bfloat16)
    wqr_h = lax.dynamic_slice(Wqr, (0, my * DR), (D, DR)).astype(jnp.bfloat16)

    out = pl.pallas_call(
        _body,
        out_shape=jax.ShapeDtypeStruct((T, D), jnp.float32),
        in_specs=[pl.BlockSpec(memory_space=pltpu.VMEM)] * 8,
        out_specs=pl.BlockSpec(memory_space=pltpu.VMEM),
        scratch_shapes=[
            pltpu.VMEM((2, T, 2 * DH), jnp.bfloat16),
            pltpu.VMEM((N_DEV - 1, T, 2 * DH), jnp.bfloat16),
            pltpu.VMEM((T, DH), jnp.bfloat16),
            pltpu.VMEM((N_DEV - 1, T, DH), jnp.bfloat16),
            pltpu.SemaphoreType.DMA((2,)),
            pltpu.SemaphoreType.DMA((N_DEV - 1,)),
            pltpu.SemaphoreType.DMA((N_DEV - 1,)),
            pltpu.SemaphoreType.DMA((N_DEV - 1,)),
        ],
        compiler_params=pltpu.CompilerParams(collective_id=0),
    )(xb, Wdkv.astype(jnp.bfloat16), Wuk.astype(jnp.bfloat16),
      Wuv.astype(jnp.bfloat16), wq_h, wqr_h,
      Wkr.astype(jnp.bfloat16), Wo.astype(jnp.bfloat16))
    return out.reshape(B, S, D)


# device time: 176893 ns/iter; 1.2910x vs baseline; 1.2910x over previous
import jax
import jax.numpy as jnp
from jax import lax
from jax.experimental import pallas as pl
from jax.experimental.pallas import tpu as pltpu

N_DEV = 16
B, S, D = 2, 512, 2048
DC = 128
H, DH, DR = 16, 128, 32
T = B * S
TH = T // 2
SCALE = (DH + DR) ** -0.5


def _body(x_ref, wdkv_ref, wuk_ref, wuv_ref, wq_ref, wqr_ref, wkr_ref,
          wo_ref, out_ref, stage_k_ref, stage_v_ref, rk_ref, rv_ref,
          o_ref, agr_ref, agl_ref,
          k_send_sems, k_recv_sems, v_send_sems, v_recv_sems,
          agr_send_sems, agr_recv_sems, agl_send_sems, agl_recv_sems):
    my = lax.axis_index("i")
    left = lax.rem(my + N_DEV - 1, N_DEV)
    right = lax.rem(my + 1, N_DEV)

    bsem = pltpu.get_barrier_semaphore()
    for nbr in (left, right):
        pl.semaphore_signal(bsem, inc=1, device_id=(nbr,),
                            device_id_type=pl.DeviceIdType.MESH)
    pl.semaphore_wait(bsem, 2)

    x = x_ref[...]
    c = jnp.dot(x, wdkv_ref[...],
                preferred_element_type=jnp.float32).astype(jnp.bfloat16)

    def local_k(j):
        return jnp.dot(c, wuk_ref[:, pl.ds(j * DH, DH)],
                       preferred_element_type=jnp.float32)

    def local_v(j):
        return jnp.dot(c, wuv_ref[:, pl.ds(j * DH, DH)],
                       preferred_element_type=jnp.float32)

    k_rdmas, v_rdmas = [], []
    for t in range(N_DEV - 1):
        p = t % 2
        if t >= 2:
            k_rdmas[t - 2].wait_send()
        kj = local_k(lax.rem(my + N_DEV - t - 1, N_DEV))
        if t >= 1:
            k_rdmas[t - 1].wait_recv()
            kj = kj + rk_ref[t - 1].astype(jnp.float32)
        stage_k_ref[p, :, :] = kj.astype(jnp.bfloat16)
        rdma = pltpu.make_async_remote_copy(
            src_ref=stage_k_ref.at[p],
            dst_ref=rk_ref.at[t],
            send_sem=k_send_sems.at[p],
            recv_sem=k_recv_sems.at[t],
            device_id=(right,),
            device_id_type=pl.DeviceIdType.MESH,
        )
        rdma.start()
        k_rdmas.append(rdma)
        if t >= 2:
            v_rdmas[t - 2].wait_send()
        vj = local_v(lax.rem(my + t + 1, N_DEV))
        if t >= 1:
            v_rdmas[t - 1].wait_recv()
            vj = vj + rv_ref[t - 1].astype(jnp.float32)
        stage_v_ref[p, :, :] = vj.astype(jnp.bfloat16)
        rdma = pltpu.make_async_remote_copy(
            src_ref=stage_v_ref.at[p],
            dst_ref=rv_ref.at[t],
            send_sem=v_send_sems.at[p],
            recv_sem=v_recv_sems.at[t],
            device_id=(left,),
            device_id_type=pl.DeviceIdType.MESH,
        )
        rdma.start()
        v_rdmas.append(rdma)

    k_rdmas[N_DEV - 2].wait_recv()
    v_rdmas[N_DEV - 2].wait_recv()
    k_h = (local_k(my) + rk_ref[N_DEV - 2].astype(jnp.float32)
           ).astype(jnp.bfloat16)
    v_h = (local_v(my) + rv_ref[N_DEV - 2].astype(jnp.float32)
           ).astype(jnp.bfloat16)

    q = jnp.dot(x, wq_ref[...],
                preferred_element_type=jnp.float32).astype(jnp.bfloat16)
    qr = jnp.dot(x, wqr_ref[...],
                 preferred_element_type=jnp.float32).astype(jnp.bfloat16)
    kr = jnp.dot(x, wkr_ref[...],
                 preferred_element_type=jnp.float32).astype(jnp.bfloat16)
    for b in range(B):
        sl = slice(b * S, (b + 1) * S)
        s1 = lax.dot_general(q[sl], k_h[sl], (((1,), (1,)), ((), ())),
                             preferred_element_type=jnp.float32)
        s2 = lax.dot_general(qr[sl], kr[sl], (((1,), (1,)), ((), ())),
                             preferred_element_type=jnp.float32)
        sc = (s1 + s2) * SCALE
        mx = jnp.max(sc, axis=1, keepdims=True)
        e = jnp.exp(sc - mx)
        pmat = (e / jnp.sum(e, axis=1, keepdims=True)).astype(jnp.bfloat16)
        ob = jnp.dot(pmat, v_h[sl], preferred_element_type=jnp.float32)
        o_ref[pl.ds(b * S, S), :] = ob.astype(jnp.bfloat16)

    def wo_rows(j):
        return wo_ref[pl.ds(j * DH, DH), :]

    agr_rdmas, agl_rdmas = [], []
    first_r = pltpu.make_async_remote_copy(
        src_ref=o_ref.at[pl.ds(0, TH)],
        dst_ref=agr_ref.at[0],
        send_sem=agr_send_sems.at[0],
        recv_sem=agr_recv_sems.at[0],
        device_id=(right,),
        device_id_type=pl.DeviceIdType.MESH,
    )
    first_r.start()
    agr_rdmas.append(first_r)
    first_l = pltpu.make_async_remote_copy(
        src_ref=o_ref.at[pl.ds(TH, TH)],
        dst_ref=agl_ref.at[0],
        send_sem=agl_send_sems.at[0],
        recv_sem=agl_recv_sems.at[0],
        device_id=(left,),
        device_id_type=pl.DeviceIdType.MESH,
    )
    first_l.start()
    agl_rdmas.append(first_l)
    wo_my = wo_rows(my)
    out_ref[pl.ds(0, TH), :] = jnp.dot(
        o_ref[pl.ds(0, TH), :], wo_my, preferred_element_type=jnp.float32)
    out_ref[pl.ds(TH, TH), :] = jnp.dot(
        o_ref[pl.ds(TH, TH), :], wo_my, preferred_element_type=jnp.float32)

    for t in range(1, N_DEV - 1):
        agr_rdmas[t - 1].wait_recv()
        rdma = pltpu.make_async_remote_copy(
            src_ref=agr_ref.at[t - 1],
            dst_ref=agr_ref.at[t],
            send_sem=agr_send_sems.at[t],
            recv_sem=agr_recv_sems.at[t],
            device_id=(right,),
            device_id_type=pl.DeviceIdType.MESH,
        )
        rdma.start()
        agr_rdmas.append(rdma)
        jr = lax.rem(my + N_DEV - t, N_DEV)
        out_ref[pl.ds(0, TH), :] += jnp.dot(
            agr_ref[t - 1], wo_rows(jr), preferred_element_type=jnp.float32)
        agl_rdmas[t - 1].wait_recv()
        rdma = pltpu.make_async_remote_copy(
            src_ref=agl_ref.at[t - 1],
            dst_ref=agl_ref.at[t],
            send_sem=agl_send_sems.at[t],
            recv_sem=agl_recv_sems.at[t],
            device_id=(left,),
            device_id_type=pl.DeviceIdType.MESH,
        )
        rdma.start()
        agl_rdmas.append(rdma)
        jl = lax.rem(my + t, N_DEV)
        out_ref[pl.ds(TH, TH), :] += jnp.dot(
            agl_ref[t - 1], wo_rows(jl), preferred_element_type=jnp.float32)

    agr_rdmas[N_DEV - 2].wait_recv()
    out_ref[pl.ds(0, TH), :] += jnp.dot(
        agr_ref[N_DEV - 2], wo_rows(right), preferred_element_type=jnp.float32)
    agl_rdmas[N_DEV - 2].wait_recv()
    out_ref[pl.ds(TH, TH), :] += jnp.dot(
        agl_ref[N_DEV - 2], wo_rows(left), preferred_element_type=jnp.float32)

    for rdmas in (k_rdmas, v_rdmas):
        rdmas[N_DEV - 3].wait_send()
        rdmas[N_DEV - 2].wait_send()
    for rdmas in (agr_rdmas, agl_rdmas):
        for rdma in rdmas:
            rdma.wait_send()


def kernel(x, Wdkv, Wuk, Wuv, Wq, Wqr, Wkr, Wo):
    my = lax.axis_index("i")
    xb = x.reshape(T, D).astype(jnp.bfloat16)
    wq_h = lax.dynamic_slice(Wq, (0, my * DH), (D, DH)).astype(jnp.bfloat16)
    wqr_h = lax.dynamic_slice(Wqr, (0, my * DR), (D, DR)).astype(jnp.bfloat16)

    out = pl.pallas_call(
        _body,
        out_shape=jax.ShapeDtypeStruct((T, D), jnp.float32),
        in_specs=[pl.BlockSpec(memory_space=pltpu.VMEM)] * 8,
        out_specs=pl.BlockSpec(memory_space=pltpu.VMEM),
        scratch_shapes=[
            pltpu.VMEM((2, T, DH), jnp.bfloat16),
            pltpu.VMEM((2, T, DH), jnp.bfloat16),
            pltpu.VMEM((N_DEV - 1, T, DH), jnp.bfloat16),
            pltpu.VMEM((N_DEV - 1, T, DH), jnp.bfloat16),
            pltpu.VMEM((T, DH), jnp.bfloat16),
            pltpu.VMEM((N_DEV - 1, TH, DH), jnp.bfloat16),
            pltpu.VMEM((N_DEV - 1, TH, DH), jnp.bfloat16),
            pltpu.SemaphoreType.DMA((2,)),
            pltpu.SemaphoreType.DMA((N_DEV - 1,)),
            pltpu.SemaphoreType.DMA((2,)),
            pltpu.SemaphoreType.DMA((N_DEV - 1,)),
            pltpu.SemaphoreType.DMA((N_DEV - 1,)),
            pltpu.SemaphoreType.DMA((N_DEV - 1,)),
            pltpu.SemaphoreType.DMA((N_DEV - 1,)),
            pltpu.SemaphoreType.DMA((N_DEV - 1,)),
        ],
        compiler_params=pltpu.CompilerParams(collective_id=0),
    )(xb, Wdkv.astype(jnp.bfloat16), Wuk.astype(jnp.bfloat16),
      Wuv.astype(jnp.bfloat16), wq_h, wqr_h,
      Wkr.astype(jnp.bfloat16), Wo.astype(jnp.bfloat16))
    return out.reshape(B, S, D)


# device time: 141840 ns/iter; 1.6100x vs baseline; 1.2471x over previous
import jax
import jax.numpy as jnp
from jax import lax
from jax.experimental import pallas as pl
from jax.experimental.pallas import tpu as pltpu

N_DEV = 16
NP = 4
B, S, D = 2, 512, 2048
DC = 128
H, DH, DR = 16, 128, 32
T = B * S
TH = T // 2
SCALE = (DH + DR) ** -0.5


def _neighbors(my):
    z = my // NP
    s = lax.rem(my, NP)
    right = NP * z + lax.rem(s + 1, NP)
    left = NP * z + lax.rem(s + NP - 1, NP)
    up = NP * lax.rem(z + 1, NP) + s
    down = NP * lax.rem(z + NP - 1, NP) + s
    return z, s, right, left, up, down


def _barrier(nbrs):
    bsem = pltpu.get_barrier_semaphore()
    for nbr in nbrs:
        pl.semaphore_signal(bsem, inc=1, device_id=(nbr,),
                            device_id_type=pl.DeviceIdType.MESH)
    pl.semaphore_wait(bsem, len(nbrs))


def _rs_attn_body(x_ref, wdkv_ref, wuk_ref, wuv_ref, wq_ref, wqr_ref,
                  wkr_ref, o_ref, st1k_ref, st1v_ref, r1k_ref, r1v_ref,
                  st2k_ref, st2v_ref, r2k_ref, r2v_ref, kb_ref, vb_ref,
                  s1k_sems, r1k_sems, s1v_sems, r1v_sems,
                  s2k_sems, r2k_sems, s2v_sems, r2v_sems):
    my = lax.axis_index("i")
    z, s, right, left, up, down = _neighbors(my)
    _barrier((right, left, up, down))

    x = x_ref[...]
    c = jnp.dot(x, wdkv_ref[...],
                preferred_element_type=jnp.float32).astype(jnp.bfloat16)

    def bundle_k(sp):
        return jnp.concatenate(
            [jnp.dot(c, wuk_ref[:, pl.ds((NP * zp + sp) * DH, DH)],
                     preferred_element_type=jnp.float32)
             for zp in range(NP)], axis=1)

    def bundle_v(sp):
        return jnp.concatenate(
            [jnp.dot(c, wuv_ref[:, pl.ds((NP * zp + sp) * DH, DH)],
                     preferred_element_type=jnp.float32)
             for zp in range(NP)], axis=1)

    k1, v1 = [], []
    for t in range(NP - 1):
        p = t % 2
        if t >= 2:
            k1[t - 2].wait_send()
        kacc = bundle_k(lax.rem(s + NP - t - 1, NP))
        if t >= 1:
            k1[t - 1].wait_recv()
            kacc = kacc + r1k_ref[t - 1].astype(jnp.float32)
        st1k_ref[p, :, :] = kacc.astype(jnp.bfloat16)
        rdma = pltpu.make_async_remote_copy(
            src_ref=st1k_ref.at[p], dst_ref=r1k_ref.at[t],
            send_sem=s1k_sems.at[p], recv_sem=r1k_sems.at[t],
            device_id=(right,), device_id_type=pl.DeviceIdType.MESH)
        rdma.start()
        k1.append(rdma)

        if t >= 2:
            v1[t - 2].wait_send()
        vacc = bundle_v(lax.rem(s + t + 1, NP))
        if t >= 1:
            v1[t - 1].wait_recv()
            vacc = vacc + r1v_ref[t - 1].astype(jnp.float32)
        st1v_ref[p, :, :] = vacc.astype(jnp.bfloat16)
        rdma = pltpu.make_async_remote_copy(
            src_ref=st1v_ref.at[p], dst_ref=r1v_ref.at[t],
            send_sem=s1v_sems.at[p], recv_sem=r1v_sems.at[t],
            device_id=(left,), device_id_type=pl.DeviceIdType.MESH)
        rdma.start()
        v1.append(rdma)

    k1[NP - 2].wait_recv()
    v1[NP - 2].wait_recv()
    kb_ref[...] = bundle_k(s) + r1k_ref[NP - 2].astype(jnp.float32)
    vb_ref[...] = bundle_v(s) + r1v_ref[NP - 2].astype(jnp.float32)

    def kb_slot(zp):
        return kb_ref[:, pl.ds(zp * DH, DH)]

    def vb_slot(zp):
        return vb_ref[:, pl.ds(zp * DH, DH)]

    k2, v2 = [], []
    for t in range(NP - 1):
        p = t % 2
        if t >= 2:
            k2[t - 2].wait_send()
        kacc = kb_slot(lax.rem(z + NP - t - 1, NP))
        if t >= 1:
            k2[t - 1].wait_recv()
            kacc = kacc + r2k_ref[t - 1].astype(jnp.float32)
        st2k_ref[p, :, :] = kacc.astype(jnp.bfloat16)
        rdma = pltpu.make_async_remote_copy(
            src_ref=st2k_ref.at[p], dst_ref=r2k_ref.at[t],
            send_sem=s2k_sems.at[p], recv_sem=r2k_sems.at[t],
            device_id=(up,), device_id_type=pl.DeviceIdType.MESH)
        rdma.start()
        k2.append(rdma)

        if t >= 2:
            v2[t - 2].wait_send()
        vacc = vb_slot(lax.rem(z + t + 1, NP))
        if t >= 1:
            v2[t - 1].wait_recv()
            vacc = vacc + r2v_ref[t - 1].astype(jnp.float32)
        st2v_ref[p, :, :] = vacc.astype(jnp.bfloat16)
        rdma = pltpu.make_async_remote_copy(
            src_ref=st2v_ref.at[p], dst_ref=r2v_ref.at[t],
            send_sem=s2v_sems.at[p], recv_sem=r2v_sems.at[t],
            device_id=(down,), device_id_type=pl.DeviceIdType.MESH)
        rdma.start()
        v2.append(rdma)

    k2[NP - 2].wait_recv()
    v2[NP - 2].wait_recv()
    k_h = (kb_slot(z) + r2k_ref[NP - 2].astype(jnp.float32)
           ).astype(jnp.bfloat16)
    v_h = (vb_slot(z) + r2v_ref[NP - 2].astype(jnp.float32)
           ).astype(jnp.bfloat16)

    q = jnp.dot(x, wq_ref[...],
                preferred_element_type=jnp.float32).astype(jnp.bfloat16)
    qr = jnp.dot(x, wqr_ref[...],
                 preferred_element_type=jnp.float32).astype(jnp.bfloat16)
    kr = jnp.dot(x, wkr_ref[...],
                 preferred_element_type=jnp.float32).astype(jnp.bfloat16)
    for b in range(B):
        sl = slice(b * S, (b + 1) * S)
        s1 = lax.dot_general(q[sl], k_h[sl], (((1,), (1,)), ((), ())),
                             preferred_element_type=jnp.float32)
        s2 = lax.dot_general(qr[sl], kr[sl], (((1,), (1,)), ((), ())),
                             preferred_element_type=jnp.float32)
        sc = (s1 + s2) * SCALE
        mx = jnp.max(sc, axis=1, keepdims=True)
        e = jnp.exp(sc - mx)
        pmat = (e / jnp.sum(e, axis=1, keepdims=True)).astype(jnp.bfloat16)
        ob = jnp.dot(pmat, v_h[sl], preferred_element_type=jnp.float32)
        o_ref[pl.ds(b * S, S), :] = ob.astype(jnp.bfloat16)

    for rdmas in (k1, v1, k2, v2):
        rdmas[NP - 3].wait_send()
        rdmas[NP - 2].wait_send()


def _ag_wo_body(o_ref, wo_ref, out_ref, bt_ref, bb_ref, zat_ref, zab_ref,
                rbt_ref, rbb_ref,
                zat_ssems, zat_rsems, zab_ssems, zab_rsems,
                bt_ssems, bt_rsems, bb_ssems, bb_rsems):
    my = lax.axis_index("i")
    z, s, right, left, up, down = _neighbors(my)
    _barrier((right, left, up, down))

    def wo_rows(j):
        return wo_ref[pl.ds(j * DH, DH), :]

    wo_my = wo_rows(my)
    out_ref[pl.ds(0, TH), :] = jnp.dot(
        o_ref[pl.ds(0, TH), :], wo_my, preferred_element_type=jnp.float32)
    out_ref[pl.ds(TH, TH), :] = jnp.dot(
        o_ref[pl.ds(TH, TH), :], wo_my, preferred_element_type=jnp.float32)

    at, ab = [], []
    for t in range(NP - 1):
        rdma = pltpu.make_async_remote_copy(
            src_ref=o_ref.at[pl.ds(0, TH)] if t == 0 else zat_ref.at[t - 1],
            dst_ref=zat_ref.at[t],
            send_sem=zat_ssems.at[t], recv_sem=zat_rsems.at[t],
            device_id=(up,), device_id_type=pl.DeviceIdType.MESH)
        if t >= 1:
            at[t - 1].wait_recv()
        rdma.start()
        at.append(rdma)
        rdma = pltpu.make_async_remote_copy(
            src_ref=o_ref.at[pl.ds(TH, TH)] if t == 0 else zab_ref.at[t - 1],
            dst_ref=zab_ref.at[t],
            send_sem=zab_ssems.at[t], recv_sem=zab_rsems.at[t],
            device_id=(down,), device_id_type=pl.DeviceIdType.MESH)
        if t >= 1:
            ab[t - 1].wait_recv()
        rdma.start()
        ab.append(rdma)
        if t >= 1:
            jt = NP * lax.rem(z + NP - t, NP) + s
            out_ref[pl.ds(0, TH), :] += jnp.dot(
                zat_ref[t - 1], wo_rows(jt),
                preferred_element_type=jnp.float32)
            jb = NP * lax.rem(z + t, NP) + s
            out_ref[pl.ds(TH, TH), :] += jnp.dot(
                zab_ref[t - 1], wo_rows(jb),
                preferred_element_type=jnp.float32)
    at[NP - 2].wait_recv()
    ab[NP - 2].wait_recv()
    jt = NP * lax.rem(z + 1, NP) + s
    out_ref[pl.ds(0, TH), :] += jnp.dot(
        zat_ref[NP - 2], wo_rows(jt), preferred_element_type=jnp.float32)
    jb = NP * lax.rem(z + NP - 1, NP) + s
    out_ref[pl.ds(TH, TH), :] += jnp.dot(
        zab_ref[NP - 2], wo_rows(jb), preferred_element_type=jnp.float32)

    bt_ref[:, pl.ds(z * DH, DH)] = o_ref[pl.ds(0, TH), :]
    bb_ref[:, pl.ds(z * DH, DH)] = o_ref[pl.ds(TH, TH), :]
    for t in range(NP - 1):
        zt = lax.rem(z + NP - t - 1, NP)
        bt_ref[:, pl.ds(zt * DH, DH)] = zat_ref[t]
        zbo = lax.rem(z + t + 1, NP)
        bb_ref[:, pl.ds(zbo * DH, DH)] = zab_ref[t]

    gt, gb = [], []
    for t in range(NP - 1):
        rdma = pltpu.make_async_remote_copy(
            src_ref=bt_ref if t == 0 else rbt_ref.at[t - 1],
            dst_ref=rbt_ref.at[t],
            send_sem=bt_ssems.at[t], recv_sem=bt_rsems.at[t],
            device_id=(right,), device_id_type=pl.DeviceIdType.MESH)
        if t >= 1:
            gt[t - 1].wait_recv()
        rdma.start()
        gt.append(rdma)
        rdma = pltpu.make_async_remote_copy(
            src_ref=bb_ref if t == 0 else rbb_ref.at[t - 1],
            dst_ref=rbb_ref.at[t],
            send_sem=bb_ssems.at[t], recv_sem=bb_rsems.at[t],
            device_id=(left,), device_id_type=pl.DeviceIdType.MESH)
        if t >= 1:
            gb[t - 1].wait_recv()
        rdma.start()
        gb.append(rdma)
        if t >= 1:
            st = lax.rem(s + NP - t, NP)
            sb = lax.rem(s + t, NP)
            for zp in range(NP):
                out_ref[pl.ds(0, TH), :] += jnp.dot(
                    rbt_ref[t - 1, :, zp * DH:(zp + 1) * DH],
                    wo_rows(NP * zp + st),
                    preferred_element_type=jnp.float32)
                out_ref[pl.ds(TH, TH), :] += jnp.dot(
                    rbb_ref[t - 1, :, zp * DH:(zp + 1) * DH],
                    wo_rows(NP * zp + sb),
                    preferred_element_type=jnp.float32)
    gt[NP - 2].wait_recv()
    gb[NP - 2].wait_recv()
    st = lax.rem(s + 1, NP)
    sb = lax.rem(s + NP - 1, NP)
    for zp in range(NP):
        out_ref[pl.ds(0, TH), :] += jnp.dot(
            rbt_ref[NP - 2, :, zp * DH:(zp + 1) * DH],
            wo_rows(NP * zp + st), preferred_element_type=jnp.float32)
        out_ref[pl.ds(TH, TH), :] += jnp.dot(
            rbb_ref[NP - 2, :, zp * DH:(zp + 1) * DH],
            wo_rows(NP * zp + sb), preferred_element_type=jnp.float32)

    for rdmas in (at, ab, gt, gb):
        for rdma in rdmas:
            rdma.wait_send()


def kernel(x, Wdkv, Wuk, Wuv, Wq, Wqr, Wkr, Wo):
    my = lax.axis_index("i")
    xb = x.reshape(T, D).astype(jnp.bfloat16)
    wq_h = lax.dynamic_slice(Wq, (0, my * DH), (D, DH)).astype(jnp.bfloat16)
    wqr_h = lax.dynamic_slice(Wqr, (0, my * DR), (D, DR)).astype(jnp.bfloat16)

    o = pl.pallas_call(
        _rs_attn_body,
        out_shape=jax.ShapeDtypeStruct((T, DH), jnp.bfloat16),
        in_specs=[pl.BlockSpec(memory_space=pltpu.VMEM)] * 7,
        out_specs=pl.BlockSpec(memory_space=pltpu.VMEM),
        scratch_shapes=[
            pltpu.VMEM((2, T, NP * DH), jnp.bfloat16),
            pltpu.VMEM((2, T, NP * DH), jnp.bfloat16),
            pltpu.VMEM((NP - 1, T, NP * DH), jnp.bfloat16),
            pltpu.VMEM((NP - 1, T, NP * DH), jnp.bfloat16),
            pltpu.VMEM((2, T, DH), jnp.bfloat16),
            pltpu.VMEM((2, T, DH), jnp.bfloat16),
            pltpu.VMEM((NP - 1, T, DH), jnp.bfloat16),
            pltpu.VMEM((NP - 1, T, DH), jnp.bfloat16),
            pltpu.VMEM((T, NP * DH), jnp.float32),
            pltpu.VMEM((T, NP * DH), jnp.float32),
            pltpu.SemaphoreType.DMA((2,)),
            pltpu.SemaphoreType.DMA((NP - 1,)),
            pltpu.SemaphoreType.DMA((2,)),
            pltpu.SemaphoreType.DMA((NP - 1,)),
            pltpu.SemaphoreType.DMA((2,)),
            pltpu.SemaphoreType.DMA((NP - 1,)),
            pltpu.SemaphoreType.DMA((2,)),
            pltpu.SemaphoreType.DMA((NP - 1,)),
        ],
        compiler_params=pltpu.CompilerParams(collective_id=0),
    )(xb, Wdkv.astype(jnp.bfloat16), Wuk.astype(jnp.bfloat16),
      Wuv.astype(jnp.bfloat16), wq_h, wqr_h, Wkr.astype(jnp.bfloat16))

    out = pl.pallas_call(
        _ag_wo_body,
        out_shape=jax.ShapeDtypeStruct((T, D), jnp.float32),
        in_specs=[pl.BlockSpec(memory_space=pltpu.VMEM)] * 2,
        out_specs=pl.BlockSpec(memory_space=pltpu.VMEM),
        scratch_shapes=[
            pltpu.VMEM((TH, NP * DH), jnp.bfloat16),
            pltpu.VMEM((TH, NP * DH), jnp.bfloat16),
            pltpu.VMEM((NP - 1, TH, DH), jnp.bfloat16),
            pltpu.VMEM((NP - 1, TH, DH), jnp.bfloat16),
            pltpu.VMEM((NP - 1, TH, NP * DH), jnp.bfloat16),
            pltpu.VMEM((NP - 1, TH, NP * DH), jnp.bfloat16),
            pltpu.SemaphoreType.DMA((NP - 1,)),
            pltpu.SemaphoreType.DMA((NP - 1,)),
            pltpu.SemaphoreType.DMA((NP - 1,)),
            pltpu.SemaphoreType.DMA((NP - 1,)),
            pltpu.SemaphoreType.DMA((NP - 1,)),
            pltpu.SemaphoreType.DMA((NP - 1,)),
            pltpu.SemaphoreType.DMA((NP - 1,)),
            pltpu.SemaphoreType.DMA((NP - 1,)),
        ],
        compiler_params=pltpu.CompilerParams(collective_id=1),
    )(o, Wo.astype(jnp.bfloat16))
    return out.reshape(B, S, D)


# device time: 133342 ns/iter; 1.7126x vs baseline; 1.0637x over previous
import jax
import jax.numpy as jnp
from jax import lax
from jax.experimental import pallas as pl
from jax.experimental.pallas import tpu as pltpu

N_DEV = 16
NP = 4
B, S, D = 2, 512, 2048
DC = 128
H, DH, DR = 16, 128, 32
T = B * S
TH = T // 2
SCALE = (DH + DR) ** -0.5


def _neighbors(my):
    z = my // NP
    s = lax.rem(my, NP)
    right = NP * z + lax.rem(s + 1, NP)
    left = NP * z + lax.rem(s + NP - 1, NP)
    up = NP * lax.rem(z + 1, NP) + s
    down = NP * lax.rem(z + NP - 1, NP) + s
    return z, s, right, left, up, down


def _barrier(nbrs):
    bsem = pltpu.get_barrier_semaphore()
    for nbr in nbrs:
        pl.semaphore_signal(bsem, inc=1, device_id=(nbr,),
                            device_id_type=pl.DeviceIdType.MESH)
    pl.semaphore_wait(bsem, len(nbrs))


def _rs_attn_body(x_ref, wdkv_ref, wuk_ref, wuv_ref, wq_ref, wqr_ref,
                  wkr_ref, o_ref, st1k_ref, st1v_ref, r1k_ref, r1v_ref,
                  st2k_ref, st2v_ref, r2k_ref, r2v_ref, kb_ref, vb_ref,
                  s1k_sems, r1k_sems, s1v_sems, r1v_sems,
                  s2k_sems, r2k_sems, s2v_sems, r2v_sems):
    my = lax.axis_index("i")
    z, s, right, left, up, down = _neighbors(my)
    _barrier((right, left, up, down))

    x = x_ref[...]
    c = jnp.dot(x, wdkv_ref[...],
                preferred_element_type=jnp.float32).astype(jnp.bfloat16)

    def bundle_k(sp):
        return jnp.concatenate(
            [jnp.dot(c, wuk_ref[:, pl.ds((NP * zp + sp) * DH, DH)],
                     preferred_element_type=jnp.float32)
             for zp in range(NP)], axis=1)

    def bundle_v(sp):
        return jnp.concatenate(
            [jnp.dot(c, wuv_ref[:, pl.ds((NP * zp + sp) * DH, DH)],
                     preferred_element_type=jnp.float32)
             for zp in range(NP)], axis=1)

    k1, v1 = [], []
    for t in range(NP - 1):
        p = t % 2
        if t >= 2:
            k1[t - 2].wait_send()
        kacc = bundle_k(lax.rem(s + NP - t - 1, NP))
        if t >= 1:
            k1[t - 1].wait_recv()
            kacc = kacc + r1k_ref[t - 1].astype(jnp.float32)
        st1k_ref[p, :, :] = kacc.astype(jnp.bfloat16)
        rdma = pltpu.make_async_remote_copy(
            src_ref=st1k_ref.at[p], dst_ref=r1k_ref.at[t],
            send_sem=s1k_sems.at[p], recv_sem=r1k_sems.at[t],
            device_id=(right,), device_id_type=pl.DeviceIdType.MESH)
        rdma.start()
        k1.append(rdma)

        if t >= 2:
            v1[t - 2].wait_send()
        vacc = bundle_v(lax.rem(s + t + 1, NP))
        if t >= 1:
            v1[t - 1].wait_recv()
            vacc = vacc + r1v_ref[t - 1].astype(jnp.float32)
        st1v_ref[p, :, :] = vacc.astype(jnp.bfloat16)
        rdma = pltpu.make_async_remote_copy(
            src_ref=st1v_ref.at[p], dst_ref=r1v_ref.at[t],
            send_sem=s1v_sems.at[p], recv_sem=r1v_sems.at[t],
            device_id=(left,), device_id_type=pl.DeviceIdType.MESH)
        rdma.start()
        v1.append(rdma)

    k1[NP - 2].wait_recv()
    v1[NP - 2].wait_recv()
    kb_ref[...] = bundle_k(s) + r1k_ref[NP - 2].astype(jnp.float32)
    vb_ref[...] = bundle_v(s) + r1v_ref[NP - 2].astype(jnp.float32)

    def kb_slot(zp):
        return kb_ref[:, pl.ds(zp * DH, DH)]

    def vb_slot(zp):
        return vb_ref[:, pl.ds(zp * DH, DH)]

    k2, v2 = [], []
    for t in range(NP - 1):
        p = t % 2
        if t >= 2:
            k2[t - 2].wait_send()
        kacc = kb_slot(lax.rem(z + NP - t - 1, NP))
        if t >= 1:
            k2[t - 1].wait_recv()
            kacc = kacc + r2k_ref[t - 1].astype(jnp.float32)
        st2k_ref[p, :, :] = kacc.astype(jnp.bfloat16)
        rdma = pltpu.make_async_remote_copy(
            src_ref=st2k_ref.at[p], dst_ref=r2k_ref.at[t],
            send_sem=s2k_sems.at[p], recv_sem=r2k_sems.at[t],
            device_id=(up,), device_id_type=pl.DeviceIdType.MESH)
        rdma.start()
        k2.append(rdma)

        if t >= 2:
            v2[t - 2].wait_send()
        vacc = vb_slot(lax.rem(z + t + 1, NP))
        if t >= 1:
            v2[t - 1].wait_recv()
            vacc = vacc + r2v_ref[t - 1].astype(jnp.float32)
        st2v_ref[p, :, :] = vacc.astype(jnp.bfloat16)
        rdma = pltpu.make_async_remote_copy(
            src_ref=st2v_ref.at[p], dst_ref=r2v_ref.at[t],
            send_sem=s2v_sems.at[p], recv_sem=r2v_sems.at[t],
            device_id=(down,), device_id_type=pl.DeviceIdType.MESH)
        rdma.start()
        v2.append(rdma)

    k2[NP - 2].wait_recv()
    v2[NP - 2].wait_recv()
    k_h = (kb_slot(z) + r2k_ref[NP - 2].astype(jnp.float32)
           ).astype(jnp.bfloat16)
    v_h = (vb_slot(z) + r2v_ref[NP - 2].astype(jnp.float32)
           ).astype(jnp.bfloat16)

    q = jnp.dot(x, wq_ref[...],
                preferred_element_type=jnp.float32).astype(jnp.bfloat16)
    qr = jnp.dot(x, wqr_ref[...],
                 preferred_element_type=jnp.float32).astype(jnp.bfloat16)
    kr = jnp.dot(x, wkr_ref[...],
                 preferred_element_type=jnp.float32).astype(jnp.bfloat16)
    for b in range(B):
        sl = slice(b * S, (b + 1) * S)
        s1 = lax.dot_general(q[sl], k_h[sl], (((1,), (1,)), ((), ())),
                             preferred_element_type=jnp.float32)
        s2 = lax.dot_general(qr[sl], kr[sl], (((1,), (1,)), ((), ())),
                             preferred_element_type=jnp.float32)
        sc = (s1 + s2) * SCALE
        mx = jnp.max(sc, axis=1, keepdims=True)
        e = jnp.exp(sc - mx)
        pmat = (e / jnp.sum(e, axis=1, keepdims=True)).astype(jnp.bfloat16)
        ob = jnp.dot(pmat, v_h[sl], preferred_element_type=jnp.float32)
        o_ref[pl.ds(b * S, S), :] = ob.astype(jnp.bfloat16)

    for rdmas in (k1, v1, k2, v2):
        rdmas[NP - 3].wait_send()
        rdmas[NP - 2].wait_send()


def _ag_wo_body(o_ref, wop_ref, out_ref, bt_ref, bb_ref, zat_ref, zab_ref,
                rbt_ref, rbb_ref,
                zat_ssems, zat_rsems, zab_ssems, zab_rsems,
                bt_ssems, bt_rsems, bb_ssems, bb_rsems):
    my = lax.axis_index("i")
    z, s, right, left, up, down = _neighbors(my)
    _barrier((right, left, up, down))

    at, ab = [], []
    for t in range(NP - 1):
        rdma = pltpu.make_async_remote_copy(
            src_ref=o_ref.at[pl.ds(0, TH)] if t == 0 else zat_ref.at[t - 1],
            dst_ref=zat_ref.at[t],
            send_sem=zat_ssems.at[t], recv_sem=zat_rsems.at[t],
            device_id=(up,), device_id_type=pl.DeviceIdType.MESH)
        if t >= 1:
            at[t - 1].wait_recv()
        rdma.start()
        at.append(rdma)
        rdma = pltpu.make_async_remote_copy(
            src_ref=o_ref.at[pl.ds(TH, TH)] if t == 0 else zab_ref.at[t - 1],
            dst_ref=zab_ref.at[t],
            send_sem=zab_ssems.at[t], recv_sem=zab_rsems.at[t],
            device_id=(down,), device_id_type=pl.DeviceIdType.MESH)
        if t >= 1:
            ab[t - 1].wait_recv()
        rdma.start()
        ab.append(rdma)

    bt_ref[:, pl.ds(z * DH, DH)] = o_ref[pl.ds(0, TH), :]
    bb_ref[:, pl.ds(z * DH, DH)] = o_ref[pl.ds(TH, TH), :]
    at[NP - 2].wait_recv()
    ab[NP - 2].wait_recv()
    for t in range(NP - 1):
        zt = lax.rem(z + NP - t - 1, NP)
        bt_ref[:, pl.ds(zt * DH, DH)] = zat_ref[t]
        zbo = lax.rem(z + t + 1, NP)
        bb_ref[:, pl.ds(zbo * DH, DH)] = zab_ref[t]

    gt, gb = [], []
    for t in range(NP - 1):
        rdma = pltpu.make_async_remote_copy(
            src_ref=bt_ref if t == 0 else rbt_ref.at[t - 1],
            dst_ref=rbt_ref.at[t],
            send_sem=bt_ssems.at[t], recv_sem=bt_rsems.at[t],
            device_id=(right,), device_id_type=pl.DeviceIdType.MESH)
        if t >= 1:
            gt[t - 1].wait_recv()
        rdma.start()
        gt.append(rdma)
        rdma = pltpu.make_async_remote_copy(
            src_ref=bb_ref if t == 0 else rbb_ref.at[t - 1],
            dst_ref=rbb_ref.at[t],
            send_sem=bb_ssems.at[t], recv_sem=bb_rsems.at[t],
            device_id=(left,), device_id_type=pl.DeviceIdType.MESH)
        if t >= 1:
            gb[t - 1].wait_recv()
        rdma.start()
        gb.append(rdma)
        if t == 0:
            out_ref[pl.ds(0, TH), :] = jnp.dot(
                bt_ref[...], wop_ref[s],
                preferred_element_type=jnp.float32)
            out_ref[pl.ds(TH, TH), :] = jnp.dot(
                bb_ref[...], wop_ref[s],
                preferred_element_type=jnp.float32)
        else:
            st = lax.rem(s + NP - t, NP)
            sb = lax.rem(s + t, NP)
            out_ref[pl.ds(0, TH), :] += jnp.dot(
                rbt_ref[t - 1], wop_ref[st],
                preferred_element_type=jnp.float32)
            out_ref[pl.ds(TH, TH), :] += jnp.dot(
                rbb_ref[t - 1], wop_ref[sb],
                preferred_element_type=jnp.float32)
    gt[NP - 2].wait_recv()
    gb[NP - 2].wait_recv()
    st = lax.rem(s + 1, NP)
    sb = lax.rem(s + NP - 1, NP)
    out_ref[pl.ds(0, TH), :] += jnp.dot(
        rbt_ref[NP - 2], wop_ref[st], preferred_element_type=jnp.float32)
    out_ref[pl.ds(TH, TH), :] += jnp.dot(
        rbb_ref[NP - 2], wop_ref[sb], preferred_element_type=jnp.float32)

    for rdmas in (at, ab, gt, gb):
        for rdma in rdmas:
            rdma.wait_send()


def kernel(x, Wdkv, Wuk, Wuv, Wq, Wqr, Wkr, Wo):
    my = lax.axis_index("i")
    xb = x.reshape(T, D).astype(jnp.bfloat16)
    wq_h = lax.dynamic_slice(Wq, (0, my * DH), (D, DH)).astype(jnp.bfloat16)
    wqr_h = lax.dynamic_slice(Wqr, (0, my * DR), (D, DR)).astype(jnp.bfloat16)

    o = pl.pallas_call(
        _rs_attn_body,
        out_shape=jax.ShapeDtypeStruct((T, DH), jnp.bfloat16),
        in_specs=[pl.BlockSpec(memory_space=pltpu.VMEM)] * 7,
        out_specs=pl.BlockSpec(memory_space=pltpu.VMEM),
        scratch_shapes=[
            pltpu.VMEM((2, T, NP * DH), jnp.bfloat16),
            pltpu.VMEM((2, T, NP * DH), jnp.bfloat16),
            pltpu.VMEM((NP - 1, T, NP * DH), jnp.bfloat16),
            pltpu.VMEM((NP - 1, T, NP * DH), jnp.bfloat16),
            pltpu.VMEM((2, T, DH), jnp.bfloat16),
            pltpu.VMEM((2, T, DH), jnp.bfloat16),
            pltpu.VMEM((NP - 1, T, DH), jnp.bfloat16),
            pltpu.VMEM((NP - 1, T, DH), jnp.bfloat16),
            pltpu.VMEM((T, NP * DH), jnp.float32),
            pltpu.VMEM((T, NP * DH), jnp.float32),
            pltpu.SemaphoreType.DMA((2,)),
            pltpu.SemaphoreType.DMA((NP - 1,)),
            pltpu.SemaphoreType.DMA((2,)),
            pltpu.SemaphoreType.DMA((NP - 1,)),
            pltpu.SemaphoreType.DMA((2,)),
            pltpu.SemaphoreType.DMA((NP - 1,)),
            pltpu.SemaphoreType.DMA((2,)),
            pltpu.SemaphoreType.DMA((NP - 1,)),
        ],
        compiler_params=pltpu.CompilerParams(collective_id=0),
    )(xb, Wdkv.astype(jnp.bfloat16), Wuk.astype(jnp.bfloat16),
      Wuv.astype(jnp.bfloat16), wq_h, wqr_h, Wkr.astype(jnp.bfloat16))

    wo_perm = (Wo.astype(jnp.bfloat16)
               .reshape(NP, NP, DH, D)
               .transpose(1, 0, 2, 3)
               .reshape(NP, NP * DH, D))

    out = pl.pallas_call(
        _ag_wo_body,
        out_shape=jax.ShapeDtypeStruct((T, D), jnp.float32),
        in_specs=[pl.BlockSpec(memory_space=pltpu.VMEM)] * 2,
        out_specs=pl.BlockSpec(memory_space=pltpu.VMEM),
        scratch_shapes=[
            pltpu.VMEM((TH, NP * DH), jnp.bfloat16),
            pltpu.VMEM((TH, NP * DH), jnp.bfloat16),
            pltpu.VMEM((NP - 1, TH, DH), jnp.bfloat16),
            pltpu.VMEM((NP - 1, TH, DH), jnp.bfloat16),
            pltpu.VMEM((NP - 1, TH, NP * DH), jnp.bfloat16),
            pltpu.VMEM((NP - 1, TH, NP * DH), jnp.bfloat16),
            pltpu.SemaphoreType.DMA((NP - 1,)),
            pltpu.SemaphoreType.DMA((NP - 1,)),
            pltpu.SemaphoreType.DMA((NP - 1,)),
            pltpu.SemaphoreType.DMA((NP - 1,)),
            pltpu.SemaphoreType.DMA((NP - 1,)),
            pltpu.SemaphoreType.DMA((NP - 1,)),
            pltpu.SemaphoreType.DMA((NP - 1,)),
            pltpu.SemaphoreType.DMA((NP - 1,)),
        ],
        compiler_params=pltpu.CompilerParams(collective_id=1),
    )(o, wo_perm)
    return out.reshape(B, S, D)


# device time: 130845 ns/iter; 1.7453x vs baseline; 1.0191x over previous
import jax
import jax.numpy as jnp
from jax import lax
from jax.experimental import pallas as pl
from jax.experimental.pallas import tpu as pltpu

N_DEV = 16
NP = 4
B, S, D = 2, 512, 2048
DC = 128
H, DH, DR = 16, 128, 32
T = B * S
TH = T // 2
BW = NP * DH
SCALE = (DH + DR) ** -0.5


def _neighbors(my):
    z = my // NP
    s = lax.rem(my, NP)
    right = NP * z + lax.rem(s + 1, NP)
    left = NP * z + lax.rem(s + NP - 1, NP)
    up = NP * lax.rem(z + 1, NP) + s
    down = NP * lax.rem(z + NP - 1, NP) + s
    return z, s, right, left, up, down


def _barrier(nbrs):
    bsem = pltpu.get_barrier_semaphore()
    for nbr in nbrs:
        pl.semaphore_signal(bsem, inc=1, device_id=(nbr,),
                            device_id_type=pl.DeviceIdType.MESH)
    pl.semaphore_wait(bsem, len(nbrs))


def _rs_body(x_ref, wdkv_ref, wukp_ref, wuvp_ref, wq_ref, wqr_ref, wkr_ref,
             kv_ref, q_ref, qr_ref, kr_ref,
             st1k_ref, st1v_ref, r1k_ref, r1v_ref,
             st2k_ref, st2v_ref, r2k_ref, r2v_ref, kb_ref, vb_ref,
             s1k_sems, r1k_sems, s1v_sems, r1v_sems,
             s2k_sems, r2k_sems, s2v_sems, r2v_sems):
    my = lax.axis_index("i")
    z, s, right, left, up, down = _neighbors(my)
    _barrier((right, left, up, down))

    x = x_ref[...]
    c = jnp.dot(x, wdkv_ref[...],
                preferred_element_type=jnp.float32).astype(jnp.bfloat16)

    def bundle_k(sp):
        return jnp.dot(c, wukp_ref[:, pl.ds(sp * BW, BW)],
                       preferred_element_type=jnp.float32)

    def bundle_v(sp):
        return jnp.dot(c, wuvp_ref[:, pl.ds(sp * BW, BW)],
                       preferred_element_type=jnp.float32)

    k1, v1 = [], []
    for t in range(NP - 1):
        p = t % 2
        if t >= 2:
            k1[t - 2].wait_send()
        kacc = bundle_k(lax.rem(s + NP - t - 1, NP))
        if t >= 1:
            k1[t - 1].wait_recv()
            kacc = kacc + r1k_ref[t - 1].astype(jnp.float32)
        st1k_ref[p, :, :] = kacc.astype(jnp.bfloat16)
        rdma = pltpu.make_async_remote_copy(
            src_ref=st1k_ref.at[p], dst_ref=r1k_ref.at[t],
            send_sem=s1k_sems.at[p], recv_sem=r1k_sems.at[t],
            device_id=(right,), device_id_type=pl.DeviceIdType.MESH)
        rdma.start()
        k1.append(rdma)

        if t >= 2:
            v1[t - 2].wait_send()
        vacc = bundle_v(lax.rem(s + t + 1, NP))
        if t >= 1:
            v1[t - 1].wait_recv()
            vacc = vacc + r1v_ref[t - 1].astype(jnp.float32)
        st1v_ref[p, :, :] = vacc.astype(jnp.bfloat16)
        rdma = pltpu.make_async_remote_copy(
            src_ref=st1v_ref.at[p], dst_ref=r1v_ref.at[t],
            send_sem=s1v_sems.at[p], recv_sem=r1v_sems.at[t],
            device_id=(left,), device_id_type=pl.DeviceIdType.MESH)
        rdma.start()
        v1.append(rdma)

        if t == 0:
            q_ref[...] = jnp.dot(
                x, wq_ref[...],
                preferred_element_type=jnp.float32).astype(jnp.bfloat16)
            qr_ref[...] = jnp.dot(
                x, wqr_ref[...],
                preferred_element_type=jnp.float32).astype(jnp.bfloat16)
            kr_ref[...] = jnp.dot(
                x, wkr_ref[...],
                preferred_element_type=jnp.float32).astype(jnp.bfloat16)

    k1[NP - 2].wait_recv()
    v1[NP - 2].wait_recv()
    kb_ref[...] = bundle_k(s) + r1k_ref[NP - 2].astype(jnp.float32)
    vb_ref[...] = bundle_v(s) + r1v_ref[NP - 2].astype(jnp.float32)

    def kb_slot(zp):
        return kb_ref[:, pl.ds(zp * DH, DH)]

    def vb_slot(zp):
        return vb_ref[:, pl.ds(zp * DH, DH)]

    k2, v2 = [], []
    for t in range(NP - 1):
        p = t % 2
        if t >= 2:
            k2[t - 2].wait_send()
        kacc = kb_slot(lax.rem(z + NP - t - 1, NP))
        if t >= 1:
            k2[t - 1].wait_recv()
            kacc = kacc + r2k_ref[t - 1].astype(jnp.float32)
        st2k_ref[p, :, :] = kacc.astype(jnp.bfloat16)
        rdma = pltpu.make_async_remote_copy(
            src_ref=st2k_ref.at[p], dst_ref=r2k_ref.at[t],
            send_sem=s2k_sems.at[p], recv_sem=r2k_sems.at[t],
            device_id=(up,), device_id_type=pl.DeviceIdType.MESH)
        rdma.start()
        k2.append(rdma)

        if t >= 2:
            v2[t - 2].wait_send()
        vacc = vb_slot(lax.rem(z + t + 1, NP))
        if t >= 1:
            v2[t - 1].wait_recv()
            vacc = vacc + r2v_ref[t - 1].astype(jnp.float32)
        st2v_ref[p, :, :] = vacc.astype(jnp.bfloat16)
        rdma = pltpu.make_async_remote_copy(
            src_ref=st2v_ref.at[p], dst_ref=r2v_ref.at[t],
            send_sem=s2v_sems.at[p], recv_sem=r2v_sems.at[t],
            device_id=(down,), device_id_type=pl.DeviceIdType.MESH)
        rdma.start()
        v2.append(rdma)

    k2[NP - 2].wait_recv()
    v2[NP - 2].wait_recv()
    kv_ref[:, 0:DH] = (kb_slot(z)
                       + r2k_ref[NP - 2].astype(jnp.float32)
                       ).astype(jnp.bfloat16)
    kv_ref[:, DH:2 * DH] = (vb_slot(z)
                            + r2v_ref[NP - 2].astype(jnp.float32)
                            ).astype(jnp.bfloat16)

    for rdmas in (k1, v1, k2, v2):
        rdmas[NP - 3].wait_send()
        rdmas[NP - 2].wait_send()


def _attn_ag_body(kv_ref, q_ref, qr_ref, kr_ref, wop_ref, out_ref,
                  o_ref, bt_ref, bb_ref, zat_ref, zab_ref, rbt_ref, rbb_ref,
                  zat_ssems, zat_rsems, zab_ssems, zab_rsems,
                  bt_ssems, bt_rsems, bb_ssems, bb_rsems):
    my = lax.axis_index("i")
    z, s, right, left, up, down = _neighbors(my)
    _barrier((right, left, up, down))

    def attention(b):
        sl = slice(b * S, (b + 1) * S)
        kh = kv_ref[sl, 0:DH]
        vh = kv_ref[sl, DH:2 * DH]
        s1 = lax.dot_general(q_ref[sl, :], kh, (((1,), (1,)), ((), ())),
                             preferred_element_type=jnp.float32)
        s2 = lax.dot_general(qr_ref[sl, :], kr_ref[sl, :],
                             (((1,), (1,)), ((), ())),
                             preferred_element_type=jnp.float32)
        sc = (s1 + s2) * SCALE
        mx = jnp.max(sc, axis=1, keepdims=True)
        e = jnp.exp(sc - mx)
        pmat = (e / jnp.sum(e, axis=1, keepdims=True)).astype(jnp.bfloat16)
        ob = jnp.dot(pmat, vh, preferred_element_type=jnp.float32)
        o_ref[pl.ds(b * S, S), :] = ob.astype(jnp.bfloat16)

    at, ab = [], []
    attention(0)
    rdma = pltpu.make_async_remote_copy(
        src_ref=o_ref.at[pl.ds(0, TH)], dst_ref=zat_ref.at[0],
        send_sem=zat_ssems.at[0], recv_sem=zat_rsems.at[0],
        device_id=(up,), device_id_type=pl.DeviceIdType.MESH)
    rdma.start()
    at.append(rdma)
    attention(1)
    rdma = pltpu.make_async_remote_copy(
        src_ref=o_ref.at[pl.ds(TH, TH)], dst_ref=zab_ref.at[0],
        send_sem=zab_ssems.at[0], recv_sem=zab_rsems.at[0],
        device_id=(down,), device_id_type=pl.DeviceIdType.MESH)
    rdma.start()
    ab.append(rdma)
    for t in range(1, NP - 1):
        rdma = pltpu.make_async_remote_copy(
            src_ref=zat_ref.at[t - 1], dst_ref=zat_ref.at[t],
            send_sem=zat_ssems.at[t], recv_sem=zat_rsems.at[t],
            device_id=(up,), device_id_type=pl.DeviceIdType.MESH)
        at[t - 1].wait_recv()
        rdma.start()
        at.append(rdma)
        rdma = pltpu.make_async_remote_copy(
            src_ref=zab_ref.at[t - 1], dst_ref=zab_ref.at[t],
            send_sem=zab_ssems.at[t], recv_sem=zab_rsems.at[t],
            device_id=(down,), device_id_type=pl.DeviceIdType.MESH)
        ab[t - 1].wait_recv()
        rdma.start()
        ab.append(rdma)

    bt_ref[:, pl.ds(z * DH, DH)] = o_ref[pl.ds(0, TH), :]
    bb_ref[:, pl.ds(z * DH, DH)] = o_ref[pl.ds(TH, TH), :]
    at[NP - 2].wait_recv()
    ab[NP - 2].wait_recv()
    for t in range(NP - 1):
        zt = lax.rem(z + NP - t - 1, NP)
        bt_ref[:, pl.ds(zt * DH, DH)] = zat_ref[t]
        zbo = lax.rem(z + t + 1, NP)
        bb_ref[:, pl.ds(zbo * DH, DH)] = zab_ref[t]

    gt, gb = [], []
    for t in range(NP - 1):
        rdma = pltpu.make_async_remote_copy(
            src_ref=bt_ref if t == 0 else rbt_ref.at[t - 1],
            dst_ref=rbt_ref.at[t],
            send_sem=bt_ssems.at[t], recv_sem=bt_rsems.at[t],
            device_id=(right,), device_id_type=pl.DeviceIdType.MESH)
        if t >= 1:
            gt[t - 1].wait_recv()
        rdma.start()
        gt.append(rdma)
        rdma = pltpu.make_async_remote_copy(
            src_ref=bb_ref if t == 0 else rbb_ref.at[t - 1],
            dst_ref=rbb_ref.at[t],
            send_sem=bb_ssems.at[t], recv_sem=bb_rsems.at[t],
            device_id=(left,), device_id_type=pl.DeviceIdType.MESH)
        if t >= 1:
            gb[t - 1].wait_recv()
        rdma.start()
        gb.append(rdma)
        if t == 0:
            out_ref[pl.ds(0, TH), :] = jnp.dot(
                bt_ref[...], wop_ref[s],
                preferred_element_type=jnp.float32)
            out_ref[pl.ds(TH, TH), :] = jnp.dot(
                bb_ref[...], wop_ref[s],
                preferred_element_type=jnp.float32)
        else:
            st = lax.rem(s + NP - t, NP)
            sb = lax.rem(s + t, NP)
            out_ref[pl.ds(0, TH), :] += jnp.dot(
                rbt_ref[t - 1], wop_ref[st],
                preferred_element_type=jnp.float32)
            out_ref[pl.ds(TH, TH), :] += jnp.dot(
                rbb_ref[t - 1], wop_ref[sb],
                preferred_element_type=jnp.float32)
    gt[NP - 2].wait_recv()
    gb[NP - 2].wait_recv()
    st = lax.rem(s + 1, NP)
    sb = lax.rem(s + NP - 1, NP)
    out_ref[pl.ds(0, TH), :] += jnp.dot(
        rbt_ref[NP - 2], wop_ref[st], preferred_element_type=jnp.float32)
    out_ref[pl.ds(TH, TH), :] += jnp.dot(
        rbb_ref[NP - 2], wop_ref[sb], preferred_element_type=jnp.float32)

    for rdmas in (at, ab, gt, gb):
        for rdma in rdmas:
            rdma.wait_send()


def kernel(x, Wdkv, Wuk, Wuv, Wq, Wqr, Wkr, Wo):
    my = lax.axis_index("i")
    xb = x.reshape(T, D).astype(jnp.bfloat16)
    wq_h = lax.dynamic_slice(Wq, (0, my * DH), (D, DH)).astype(jnp.bfloat16)
    wqr_h = lax.dynamic_slice(Wqr, (0, my * DR), (D, DR)).astype(jnp.bfloat16)

    def col_perm(w):
        return (w.astype(jnp.bfloat16)
                .reshape(DC, NP, NP, DH)
                .transpose(0, 2, 1, 3)
                .reshape(DC, H * DH))

    kv, q, qr, kr = pl.pallas_call(
        _rs_body,
        out_shape=[
            jax.ShapeDtypeStruct((T, 2 * DH), jnp.bfloat16),
            jax.ShapeDtypeStruct((T, DH), jnp.bfloat16),
            jax.ShapeDtypeStruct((T, DR), jnp.bfloat16),
            jax.ShapeDtypeStruct((T, DR), jnp.bfloat16),
        ],
        in_specs=[pl.BlockSpec(memory_space=pltpu.VMEM)] * 7,
        out_specs=[pl.BlockSpec(memory_space=pltpu.VMEM)] * 4,
        scratch_shapes=[
            pltpu.VMEM((2, T, BW), jnp.bfloat16),
            pltpu.VMEM((2, T, BW), jnp.bfloat16),
            pltpu.VMEM((NP - 1, T, BW), jnp.bfloat16),
            pltpu.VMEM((NP - 1, T, BW), jnp.bfloat16),
            pltpu.VMEM((2, T, DH), jnp.bfloat16),
            pltpu.VMEM((2, T, DH), jnp.bfloat16),
            pltpu.VMEM((NP - 1, T, DH), jnp.bfloat16),
            pltpu.VMEM((NP - 1, T, DH), jnp.bfloat16),
            pltpu.VMEM((T, BW), jnp.float32),
            pltpu.VMEM((T, BW), jnp.float32),
            pltpu.SemaphoreType.DMA((2,)),
            pltpu.SemaphoreType.DMA((NP - 1,)),
            pltpu.SemaphoreType.DMA((2,)),
            pltpu.SemaphoreType.DMA((NP - 1,)),
            pltpu.SemaphoreType.DMA((2,)),
            pltpu.SemaphoreType.DMA((NP - 1,)),
            pltpu.SemaphoreType.DMA((2,)),
            pltpu.SemaphoreType.DMA((NP - 1,)),
        ],
        compiler_params=pltpu.CompilerParams(collective_id=0),
    )(xb, Wdkv.astype(jnp.bfloat16), col_perm(Wuk), col_perm(Wuv),
      wq_h, wqr_h, Wkr.astype(jnp.bfloat16))

    wo_perm = (Wo.astype(jnp.bfloat16)
               .reshape(NP, NP, DH, D)
               .transpose(1, 0, 2, 3)
               .reshape(NP, BW, D))

    out = pl.pallas_call(
        _attn_ag_body,
        out_shape=jax.ShapeDtypeStruct((T, D), jnp.float32),
        in_specs=[pl.BlockSpec(memory_space=pltpu.VMEM)] * 5,
        out_specs=pl.BlockSpec(memory_space=pltpu.VMEM),
        scratch_shapes=[
            pltpu.VMEM((T, DH), jnp.bfloat16),
            pltpu.VMEM((TH, BW), jnp.bfloat16),
            pltpu.VMEM((TH, BW), jnp.bfloat16),
            pltpu.VMEM((NP - 1, TH, DH), jnp.bfloat16),
            pltpu.VMEM((NP - 1, TH, DH), jnp.bfloat16),
            pltpu.VMEM((NP - 1, TH, BW), jnp.bfloat16),
            pltpu.VMEM((NP - 1, TH, BW), jnp.bfloat16),
            pltpu.SemaphoreType.DMA((NP - 1,)),
            pltpu.SemaphoreType.DMA((NP - 1,)),
            pltpu.SemaphoreType.DMA((NP - 1,)),
            pltpu.SemaphoreType.DMA((NP - 1,)),
            pltpu.SemaphoreType.DMA((NP - 1,)),
            pltpu.SemaphoreType.DMA((NP - 1,)),
            pltpu.SemaphoreType.DMA((NP - 1,)),
            pltpu.SemaphoreType.DMA((NP - 1,)),
        ],
        compiler_params=pltpu.CompilerParams(collective_id=1),
    )(kv, q, qr, kr, wo_perm)
    return out.reshape(B, S, D)
